# baseline (speedup 1.0000x reference)
import functools

import jax
import jax.numpy as jnp
from jax import lax
from jax.experimental import pallas as pl
from jax.experimental.pallas import tpu as pltpu
from jax.experimental.pallas import tpu_sc as plsc

_NC = 2
_NS = 16
_C = 128
_ZR = 128
_DEG_W = 8


def _sc_mesh():
    return plsc.VectorSubcoreMesh(
        core_axis_name="c", subcore_axis_name="s",
        num_cores=_NC, num_subcores=_NS)


_SC_PARAMS = pltpu.CompilerParams(use_tc_tiling_on_sc=False)


_D0 = 8


def _chunk_split(e):
    k0 = pl.cdiv(pl.cdiv(e, _C), _NS)
    k0 = pl.cdiv(k0, _D0) * _D0
    return k0


def _sc_deg_call(col_ch, ones, zeros, n_pad, kd):
    rpt = n_pad // _NS

    @functools.partial(
        pl.kernel,
        out_type=jax.ShapeDtypeStruct((_NC, n_pad, _DEG_W), jnp.float32),
        mesh=_sc_mesh(),
        compiler_params=_SC_PARAMS,
        scratch_types=[
            pltpu.VMEM((kd, _C), jnp.int32),
            pltpu.VMEM((_C, _DEG_W), jnp.float32),
            pltpu.VMEM((_ZR, _DEG_W), jnp.float32),
            pltpu.VMEM_SHARED((n_pad, _DEG_W), jnp.float32),
        ],
    )
    def deg_kernel(col_hbm, ones_hbm, zeros_hbm, out_hbm, cid_v, ones_v, zbuf_v, acc):
        c = lax.axis_index("c")
        s = lax.axis_index("s")
        wid = c * _NS + s
        base = s * rpt
        pltpu.sync_copy(zeros_hbm, zbuf_v)

        def zstep(z, carry):
            pltpu.sync_copy(zbuf_v, acc.at[pl.ds(base + z * _ZR, _ZR)])
            return carry
        lax.fori_loop(0, rpt // _ZR, zstep, 0)

        pltpu.sync_copy(ones_hbm, ones_v)
        pltpu.sync_copy(col_hbm.at[pl.ds(wid * kd, kd)], cid_v)
        plsc.subcore_barrier()

        def step(j, carry):
            pltpu.sync_copy(ones_v, acc.at[cid_v.at[j]], add=True)
            return carry
        lax.fori_loop(0, kd, step, 0)
        plsc.subcore_barrier()

        def wstep(z, carry):
            ds = pl.ds(base + z * _ZR, _ZR)
            pltpu.sync_copy(acc.at[ds], zbuf_v)
            pltpu.sync_copy(zbuf_v, out_hbm.at[c, ds])
            return carry
        lax.fori_loop(0, rpt // _ZR, wstep, 0)

    return deg_kernel(col_ch, ones, zeros)


def _sc_agg_call(table, row_ch, col_ch, zeros, n_pad, k0):
    h = table.shape[1]
    rpt = n_pad // _NS

    @functools.partial(
        pl.kernel,
        out_type=jax.ShapeDtypeStruct((1, n_pad, h), jnp.float32),
        mesh=_sc_mesh(),
        compiler_params=_SC_PARAMS,
        scratch_types=[
            pltpu.VMEM((k0 + _D0, _C), jnp.int32),
            pltpu.VMEM((k0 + _D0, _C), jnp.int32),
            [pltpu.VMEM((_C, h), jnp.float32) for _ in range(_D0)],
            pltpu.VMEM((_ZR, h), jnp.float32),
            pltpu.VMEM_SHARED((n_pad, h), jnp.float32),
            [pltpu.SemaphoreType.DMA for _ in range(_D0)],
        ],
    )
    def agg_kernel(tab_hbm, row_hbm, col_hbm, zeros_hbm, out_hbm,
                   rid_v, cid_v, rows, zbuf_v, acc, sems):
        c = lax.axis_index("c")
        s = lax.axis_index("s")

        @pl.when(c == 0)
        def _():
            base = s * rpt
            pltpu.sync_copy(zeros_hbm, zbuf_v)

            def zstep(z, carry):
                pltpu.sync_copy(zbuf_v, acc.at[pl.ds(base + z * _ZR, _ZR)])
                return carry
            lax.fori_loop(0, rpt // _ZR, zstep, 0)

            start = s * k0
            pltpu.sync_copy(row_hbm.at[pl.ds(start, k0 + _D0)], rid_v)
            pltpu.sync_copy(col_hbm.at[pl.ds(start, k0 + _D0)], cid_v)

            plsc.subcore_barrier()

            for b in range(_D0):
                pltpu.async_copy(tab_hbm.at[rid_v.at[b]], rows[b], sems[b])

            def step(i, carry):
                for b in range(_D0):
                    j = i * _D0 + b
                    pltpu.make_async_copy(
                        tab_hbm.at[rid_v.at[j]], rows[b], sems[b]).wait()
                    pltpu.sync_copy(rows[b], acc.at[cid_v.at[j]], add=True)
                    pltpu.async_copy(
                        tab_hbm.at[rid_v.at[j + _D0]], rows[b], sems[b])
                return carry
            lax.fori_loop(0, k0 // _D0, step, 0)

            for b in range(_D0):
                pltpu.make_async_copy(
                    tab_hbm.at[rid_v.at[0]], rows[b], sems[b]).wait()

            plsc.subcore_barrier()

            def wstep(z, carry):
                ds = pl.ds(base + z * _ZR, _ZR)
                pltpu.sync_copy(acc.at[ds], zbuf_v)
                pltpu.sync_copy(zbuf_v, out_hbm.at[0, ds])
                return carry
            lax.fori_loop(0, rpt // _ZR, wstep, 0)

    return agg_kernel(table, row_ch, col_ch, zeros)


def _dinv_from(degp, n):
    deg = degp[0, :n, :1] + degp[1, :n, :1]
    return jnp.where(deg > 0.0, lax.rsqrt(deg), 0.0)


def _tc_pass0(x, w1i, w1r):
    n, _ = x.shape
    h = w1i.shape[1]

    def body(x_ref, w1i_ref, w1r_ref, t1_ref, r1_ref):
        xb = x_ref[...]
        t1_ref[...] = jnp.dot(xb, w1i_ref[...], preferred_element_type=jnp.float32)
        r1_ref[...] = jnp.dot(xb, w1r_ref[...], preferred_element_type=jnp.float32)

    return pl.pallas_call(
        body,
        out_shape=[
            jax.ShapeDtypeStruct((n, h), jnp.float32),
            jax.ShapeDtypeStruct((n, h), jnp.float32),
        ],
    )(x, w1i, w1r)


def _tc_pass1b(t1, degp):
    n, h = t1.shape

    def body(t1_ref, degp_ref, t1s_ref):
        dinv = _dinv_from(degp_ref[...], n)
        t1s_ref[...] = t1_ref[...] * dinv

    return pl.pallas_call(
        body,
        out_shape=jax.ShapeDtypeStruct((n, h), jnp.float32),
    )(t1, degp)


def _tc_pass2(degp, p1, r1, b1, w2i, w2r):
    n, h = r1.shape

    def body(degp_ref, p1_ref, r1_ref, b1_ref, w2i_ref, w2r_ref, t2s_ref, r2_ref):
        dinv = _dinv_from(degp_ref[...], n)
        agg = p1_ref[0, :n] * dinv
        h1 = jnp.maximum(agg + r1_ref[...] + b1_ref[...], 0.0)
        t = jnp.dot(h1, w2i_ref[...], preferred_element_type=jnp.float32)
        t2s_ref[...] = t * dinv
        r2_ref[...] = jnp.dot(h1, w2r_ref[...], preferred_element_type=jnp.float32)

    return pl.pallas_call(
        body,
        out_shape=[
            jax.ShapeDtypeStruct((n, h), jnp.float32),
            jax.ShapeDtypeStruct((n, h), jnp.float32),
        ],
    )(degp, p1, r1, b1, w2i, w2r)


def _tc_pass3(degp, p2, r2, b2, fcw_t, fcb):
    n, h = r2.shape

    def body(degp_ref, p2_ref, r2_ref, b2_ref, fcw_ref, fcb_ref, out_ref):
        dinv = _dinv_from(degp_ref[...], n)
        agg = p2_ref[0, :n] * dinv
        h2 = jnp.maximum(agg + r2_ref[...] + b2_ref[...], 0.0)
        out = jnp.dot(h2, fcw_ref[...], preferred_element_type=jnp.float32)
        out_ref[...] = out + fcb_ref[0, 0]

    return pl.pallas_call(
        body,
        out_shape=jax.ShapeDtypeStruct((n, 1), jnp.float32),
    )(degp, p2, r2, b2, fcw_t, fcb)


def kernel(x, edge_index, init_w1, root_w1, bias1, init_w2, root_w2, bias2, fc_w, fc_b):
    n, _ = x.shape
    h = init_w1.shape[-1]
    e = edge_index.shape[1]

    n_pad = pl.cdiv(n, _NS * _ZR) * (_NS * _ZR)
    k0 = _chunk_split(e)
    kd = k0 // 2
    totc = _NS * k0
    e_pad = (totc + _D0) * _C

    row = jnp.concatenate(
        [edge_index[0], jnp.zeros((e_pad - e,), jnp.int32)])
    col = jnp.concatenate(
        [edge_index[1], jnp.full((e_pad - e,), n_pad - 1, jnp.int32)])
    row_ch = row.reshape(totc + _D0, _C)
    col_ch = col.reshape(totc + _D0, _C)

    ones = jnp.ones((_C, _DEG_W), jnp.float32)
    zeros_deg = jnp.zeros((_ZR, _DEG_W), jnp.float32)
    zeros_h = jnp.zeros((_ZR, h), jnp.float32)

    w1i = init_w1[0]
    w1r = root_w1[0, 0]
    b1 = bias1.reshape(1, h)
    w2i = init_w2[0]
    w2r = root_w2[0, 0]
    b2 = bias2.reshape(1, h)
    fcw_t = fc_w.reshape(1, h).T
    fcb = fc_b.reshape(1, 1)

    t1, r1 = _tc_pass0(x, w1i, w1r)
    degp = _sc_deg_call(col_ch, ones, zeros_deg, n_pad, kd)
    t1s = _tc_pass1b(t1, degp)
    p1 = _sc_agg_call(t1s, row_ch, col_ch, zeros_h, n_pad, k0)
    t2s, r2 = _tc_pass2(degp, p1, r1, b1, w2i, w2r)
    p2 = _sc_agg_call(t2s, row_ch, col_ch, zeros_h, n_pad, k0)
    return _tc_pass3(degp, p2, r2, b2, fcw_t, fcb)


# speedup vs baseline: 2.1354x; 2.1354x over previous
import functools

import jax
import jax.numpy as jnp
from jax import lax
from jax.experimental import pallas as pl
from jax.experimental.pallas import tpu as pltpu
from jax.experimental.pallas import tpu_sc as plsc

_NC = 2
_NS = 16
_C = 128
_ZR = 128
_DEG_W = 8


def _sc_mesh():
    return plsc.VectorSubcoreMesh(
        core_axis_name="c", subcore_axis_name="s",
        num_cores=_NC, num_subcores=_NS)


_SC_PARAMS = pltpu.CompilerParams(use_tc_tiling_on_sc=False)


_D0 = 8


def _chunk_split(e):
    ku = pl.cdiv(pl.cdiv(e, _C), _NC * _NS)
    return pl.cdiv(ku, _D0) * _D0


def _sc_deg_call(col_ch, ones, zeros, n_pad, kd):
    rpt = n_pad // _NS

    @functools.partial(
        pl.kernel,
        out_type=jax.ShapeDtypeStruct((_NC, n_pad, _DEG_W), jnp.float32),
        mesh=_sc_mesh(),
        compiler_params=_SC_PARAMS,
        scratch_types=[
            pltpu.VMEM((kd, _C), jnp.int32),
            pltpu.VMEM((_C, _DEG_W), jnp.float32),
            pltpu.VMEM((_ZR, _DEG_W), jnp.float32),
            pltpu.VMEM_SHARED((n_pad, _DEG_W), jnp.float32),
        ],
    )
    def deg_kernel(col_hbm, ones_hbm, zeros_hbm, out_hbm, cid_v, ones_v, zbuf_v, acc):
        c = lax.axis_index("c")
        s = lax.axis_index("s")
        wid = c * _NS + s
        base = s * rpt
        pltpu.sync_copy(zeros_hbm, zbuf_v)

        def zstep(z, carry):
            pltpu.sync_copy(zbuf_v, acc.at[pl.ds(base + z * _ZR, _ZR)])
            return carry
        lax.fori_loop(0, rpt // _ZR, zstep, 0)

        pltpu.sync_copy(ones_hbm, ones_v)
        pltpu.sync_copy(col_hbm.at[pl.ds(wid * kd, kd)], cid_v)
        plsc.subcore_barrier()

        def step(j, carry):
            pltpu.sync_copy(ones_v, acc.at[cid_v.at[j]], add=True)
            return carry
        lax.fori_loop(0, kd, step, 0)
        plsc.subcore_barrier()

        def wstep(z, carry):
            ds = pl.ds(base + z * _ZR, _ZR)
            pltpu.sync_copy(acc.at[ds], zbuf_v)
            pltpu.sync_copy(zbuf_v, out_hbm.at[c, ds])
            return carry
        lax.fori_loop(0, rpt // _ZR, wstep, 0)

    return deg_kernel(col_ch, ones, zeros)


def _sc_agg_call(table, row_ch, col_ch, zeros, n_pad, ku):
    h = table.shape[1]
    rpt = n_pad // _NS

    @functools.partial(
        pl.kernel,
        out_type=jax.ShapeDtypeStruct((_NC, n_pad, h), jnp.float32),
        mesh=_sc_mesh(),
        compiler_params=_SC_PARAMS,
        scratch_types=[
            pltpu.VMEM((ku + _D0, _C), jnp.int32),
            pltpu.VMEM((ku + _D0, _C), jnp.int32),
            [pltpu.VMEM((_C, h), jnp.float32) for _ in range(_D0)],
            pltpu.VMEM((_ZR, h), jnp.float32),
            pltpu.VMEM_SHARED((n_pad, h), jnp.float32),
            pltpu.VMEM_SHARED((n_pad, h), jnp.float32),
            [pltpu.SemaphoreType.DMA for _ in range(_D0)],
        ],
    )
    def agg_kernel(tab_hbm, row_hbm, col_hbm, zeros_hbm, out_hbm,
                   rid_v, cid_v, rows, zbuf_v, acc, tabs, sems):
        c = lax.axis_index("c")
        s = lax.axis_index("s")
        base = s * rpt
        pltpu.sync_copy(zeros_hbm, zbuf_v)

        def zstep(z, carry):
            pltpu.sync_copy(zbuf_v, acc.at[pl.ds(base + z * _ZR, _ZR)])
            return carry
        lax.fori_loop(0, rpt // _ZR, zstep, 0)

        def tstep(z, carry):
            ds = pl.ds(base + z * _ZR, _ZR)
            pltpu.sync_copy(tab_hbm.at[ds], rows[0])
            pltpu.sync_copy(rows[0], tabs.at[ds])
            return carry
        lax.fori_loop(0, rpt // _ZR, tstep, 0)

        start = (c * _NS + s) * ku
        pltpu.sync_copy(row_hbm.at[pl.ds(start, ku + _D0)], rid_v)
        pltpu.sync_copy(col_hbm.at[pl.ds(start, ku + _D0)], cid_v)

        plsc.subcore_barrier()

        for b in range(_D0):
            pltpu.async_copy(tabs.at[rid_v.at[b]], rows[b], sems[b])

        def step(i, carry):
            for b in range(_D0):
                j = i * _D0 + b
                pltpu.make_async_copy(
                    tabs.at[rid_v.at[j]], rows[b], sems[b]).wait()
                pltpu.sync_copy(rows[b], acc.at[cid_v.at[j]], add=True)
                pltpu.async_copy(
                    tabs.at[rid_v.at[j + _D0]], rows[b], sems[b])
            return carry
        lax.fori_loop(0, ku // _D0, step, 0)

        for b in range(_D0):
            pltpu.make_async_copy(
                tabs.at[rid_v.at[0]], rows[b], sems[b]).wait()

        plsc.subcore_barrier()

        def wstep(z, carry):
            ds = pl.ds(base + z * _ZR, _ZR)
            pltpu.sync_copy(acc.at[ds], zbuf_v)
            pltpu.sync_copy(zbuf_v, out_hbm.at[c, ds])
            return carry
        lax.fori_loop(0, rpt // _ZR, wstep, 0)

    return agg_kernel(table, row_ch, col_ch, zeros)


def _dinv_from(degp, n):
    deg = degp[0, :n, :1] + degp[1, :n, :1]
    return jnp.where(deg > 0.0, lax.rsqrt(deg), 0.0)


def _tc_pass0(x, w1i, w1r):
    n, _ = x.shape
    h = w1i.shape[1]

    def body(x_ref, w1i_ref, w1r_ref, t1_ref, r1_ref):
        xb = x_ref[...]
        t1_ref[...] = jnp.dot(xb, w1i_ref[...], preferred_element_type=jnp.float32)
        r1_ref[...] = jnp.dot(xb, w1r_ref[...], preferred_element_type=jnp.float32)

    return pl.pallas_call(
        body,
        out_shape=[
            jax.ShapeDtypeStruct((n, h), jnp.float32),
            jax.ShapeDtypeStruct((n, h), jnp.float32),
        ],
    )(x, w1i, w1r)


def _tc_pass1b(t1, degp):
    n, h = t1.shape

    def body(t1_ref, degp_ref, t1s_ref):
        dinv = _dinv_from(degp_ref[...], n)
        t1s_ref[...] = t1_ref[...] * dinv

    return pl.pallas_call(
        body,
        out_shape=jax.ShapeDtypeStruct((n, h), jnp.float32),
    )(t1, degp)


def _tc_pass2(degp, p1, r1, b1, w2i, w2r):
    n, h = r1.shape

    def body(degp_ref, p1_ref, r1_ref, b1_ref, w2i_ref, w2r_ref, t2s_ref, r2_ref):
        dinv = _dinv_from(degp_ref[...], n)
        p = p1_ref[...]
        agg = (p[0, :n] + p[1, :n]) * dinv
        h1 = jnp.maximum(agg + r1_ref[...] + b1_ref[...], 0.0)
        t = jnp.dot(h1, w2i_ref[...], preferred_element_type=jnp.float32)
        t2s_ref[...] = t * dinv
        r2_ref[...] = jnp.dot(h1, w2r_ref[...], preferred_element_type=jnp.float32)

    return pl.pallas_call(
        body,
        out_shape=[
            jax.ShapeDtypeStruct((n, h), jnp.float32),
            jax.ShapeDtypeStruct((n, h), jnp.float32),
        ],
    )(degp, p1, r1, b1, w2i, w2r)


def _tc_pass3(degp, p2, r2, b2, fcw_t, fcb):
    n, h = r2.shape

    def body(degp_ref, p2_ref, r2_ref, b2_ref, fcw_ref, fcb_ref, out_ref):
        dinv = _dinv_from(degp_ref[...], n)
        p = p2_ref[...]
        agg = (p[0, :n] + p[1, :n]) * dinv
        h2 = jnp.maximum(agg + r2_ref[...] + b2_ref[...], 0.0)
        out = jnp.dot(h2, fcw_ref[...], preferred_element_type=jnp.float32)
        out_ref[...] = out + fcb_ref[0, 0]

    return pl.pallas_call(
        body,
        out_shape=jax.ShapeDtypeStruct((n, 1), jnp.float32),
    )(degp, p2, r2, b2, fcw_t, fcb)


def kernel(x, edge_index, init_w1, root_w1, bias1, init_w2, root_w2, bias2, fc_w, fc_b):
    n, _ = x.shape
    h = init_w1.shape[-1]
    e = edge_index.shape[1]

    n_pad = pl.cdiv(n, _NS * _ZR) * (_NS * _ZR)
    ku = _chunk_split(e)
    kd = ku
    totc = _NC * _NS * ku
    e_pad = (totc + _D0) * _C

    row = jnp.concatenate(
        [edge_index[0], jnp.zeros((e_pad - e,), jnp.int32)])
    col = jnp.concatenate(
        [edge_index[1], jnp.full((e_pad - e,), n_pad - 1, jnp.int32)])
    row_ch = row.reshape(totc + _D0, _C)
    col_ch = col.reshape(totc + _D0, _C)

    ones = jnp.ones((_C, _DEG_W), jnp.float32)
    zeros_deg = jnp.zeros((_ZR, _DEG_W), jnp.float32)
    zeros_h = jnp.zeros((_ZR, h), jnp.float32)

    w1i = init_w1[0]
    w1r = root_w1[0, 0]
    b1 = bias1.reshape(1, h)
    w2i = init_w2[0]
    w2r = root_w2[0, 0]
    b2 = bias2.reshape(1, h)
    fcw_t = fc_w.reshape(1, h).T
    fcb = fc_b.reshape(1, 1)

    pad_n = ((0, n_pad - n), (0, 0))
    t1, r1 = _tc_pass0(x, w1i, w1r)
    degp = _sc_deg_call(col_ch, ones, zeros_deg, n_pad, kd)
    t1s = jnp.pad(_tc_pass1b(t1, degp), pad_n)
    p1 = _sc_agg_call(t1s, row_ch, col_ch, zeros_h, n_pad, ku)
    t2s, r2 = _tc_pass2(degp, p1, r1, b1, w2i, w2r)
    p2 = _sc_agg_call(jnp.pad(t2s, pad_n), row_ch, col_ch, zeros_h, n_pad, ku)
    return _tc_pass3(degp, p2, r2, b2, fcw_t, fcb)
